# baseline (speedup 1.0000x reference)
import functools

import jax
import jax.numpy as jnp
from jax.experimental import pallas as pl
from jax.experimental.pallas import tpu as pltpu

TOP_K = 64
NEG_BIG = -1000000000.0


def _select_kernel(x_ref, w_impT_ref, b_imp_ref, wk_ref, bk_ref, wv_ref,
                   bv_ref, idx_ref, kproj_ref, vproj_ref, sel_ref, *, k):
    S, D = x_ref.shape
    xb = x_ref[...]
    scores = jax.lax.dot_general(
        w_impT_ref[...], xb,
        dimension_numbers=(((1,), (1,)), ((), ())),
        preferred_element_type=jnp.float32) + b_imp_ref[...]
    lane_iota = jax.lax.broadcasted_iota(jnp.int32, (1, S), 1)
    k_iota = jax.lax.broadcasted_iota(jnp.int32, (1, k), 1)

    def body(i, carry):
        sc, acc = carry
        m = jnp.max(sc)
        cand = jnp.where(sc == m, lane_iota, S)
        idx = jnp.min(cand).astype(jnp.int32)
        acc = jnp.where(k_iota == i, idx, acc)
        sel_ref[pl.ds(i, 1), :] = x_ref[pl.ds(idx, 1), :]
        sc = jnp.where(lane_iota == idx, -jnp.inf, sc)
        return sc, acc

    acc0 = jnp.zeros((1, k), dtype=jnp.int32)
    _, acc = jax.lax.fori_loop(0, k, body, (scores, acc0))
    idx_ref[...] = acc
    sel = sel_ref[...]
    kproj_ref[...] = jnp.dot(sel, wk_ref[...],
                             preferred_element_type=jnp.float32) + bk_ref[...]
    vproj_ref[...] = jnp.dot(sel, wv_ref[...],
                             preferred_element_type=jnp.float32) + bv_ref[...]


def _attn_kernel(x_ref, wq_ref, bq_ref, kproj_ref, vproj_ref, idx_ref,
                 wo_ref, bo_ref, out_ref, *, heads, dh, tile, k):
    t = pl.program_id(1)
    xt = x_ref[...]
    q = jnp.dot(xt, wq_ref[...],
                preferred_element_type=jnp.float32) + bq_ref[...]
    kp = kproj_ref[...]
    vp = vproj_ref[...]
    kpos = idx_ref[...]
    qpos = t * tile + jax.lax.broadcasted_iota(jnp.int32, (tile, 1), 0)
    mask = qpos >= kpos
    scale = 1.0 / jnp.sqrt(jnp.asarray(dh, dtype=jnp.float32))
    ctx = []
    for h in range(heads):
        sl = slice(h * dh, (h + 1) * dh)
        qh = q[:, sl]
        sh = jax.lax.dot_general(
            qh, kp[:, sl],
            dimension_numbers=(((1,), (1,)), ((), ())),
            preferred_element_type=jnp.float32) * scale
        sh = jnp.where(mask, sh, NEG_BIG)
        m = jnp.max(sh, axis=1, keepdims=True)
        e = jnp.exp(sh - m)
        attn = e / jnp.sum(e, axis=1, keepdims=True)
        ctx.append(jnp.dot(attn, vp[:, sl],
                           preferred_element_type=jnp.float32))
    ctx = jnp.concatenate(ctx, axis=1)
    out_ref[...] = jnp.dot(ctx, wo_ref[...],
                           preferred_element_type=jnp.float32) + bo_ref[...]


def kernel(x, W_imp, b_imp, Wq, bq, Wk, bk, Wv, bv, Wo, bo):
    B, S, D = x.shape
    HD = Wq.shape[1]
    heads, dh = 16, HD // 16
    k = min(TOP_K, S)
    tile = 512
    nt = S // tile

    w_impT = W_imp.T
    b_imp2 = b_imp.reshape(1, 1)
    bq2, bk2, bv2 = bq.reshape(1, HD), bk.reshape(1, HD), bv.reshape(1, HD)
    bo2 = bo.reshape(1, D)

    idx3, kproj, vproj = pl.pallas_call(
        functools.partial(_select_kernel, k=k),
        grid=(B,),
        in_specs=[
            pl.BlockSpec((None, S, D), lambda b: (b, 0, 0)),
            pl.BlockSpec((1, D), lambda b: (0, 0)),
            pl.BlockSpec((1, 1), lambda b: (0, 0)),
            pl.BlockSpec((D, HD), lambda b: (0, 0)),
            pl.BlockSpec((1, HD), lambda b: (0, 0)),
            pl.BlockSpec((D, HD), lambda b: (0, 0)),
            pl.BlockSpec((1, HD), lambda b: (0, 0)),
        ],
        out_specs=[
            pl.BlockSpec((None, 1, k), lambda b: (b, 0, 0)),
            pl.BlockSpec((None, k, HD), lambda b: (b, 0, 0)),
            pl.BlockSpec((None, k, HD), lambda b: (b, 0, 0)),
        ],
        out_shape=[
            jax.ShapeDtypeStruct((B, 1, k), jnp.int32),
            jax.ShapeDtypeStruct((B, k, HD), jnp.float32),
            jax.ShapeDtypeStruct((B, k, HD), jnp.float32),
        ],
        scratch_shapes=[pltpu.VMEM((k, D), jnp.float32)],
    )(x, w_impT, b_imp2, Wk, bk2, Wv, bv2)

    out = pl.pallas_call(
        functools.partial(_attn_kernel, heads=heads, dh=dh, tile=tile, k=k),
        grid=(B, nt),
        in_specs=[
            pl.BlockSpec((None, tile, D), lambda b, t: (b, t, 0)),
            pl.BlockSpec((D, HD), lambda b, t: (0, 0)),
            pl.BlockSpec((1, HD), lambda b, t: (0, 0)),
            pl.BlockSpec((None, k, HD), lambda b, t: (b, 0, 0)),
            pl.BlockSpec((None, k, HD), lambda b, t: (b, 0, 0)),
            pl.BlockSpec((None, 1, k), lambda b, t: (b, 0, 0)),
            pl.BlockSpec((HD, D), lambda b, t: (0, 0)),
            pl.BlockSpec((1, D), lambda b, t: (0, 0)),
        ],
        out_specs=pl.BlockSpec((None, tile, D), lambda b, t: (b, t, 0)),
        out_shape=jax.ShapeDtypeStruct((B, S, D), jnp.float32),
    )(x, Wq, bq2, kproj, vproj, idx3, Wo, bo2)

    return out, idx3.reshape(B, k)


# speedup vs baseline: 1.0345x; 1.0345x over previous
import functools

import jax
import jax.numpy as jnp
from jax.experimental import pallas as pl
from jax.experimental.pallas import tpu as pltpu

TOP_K = 64
NEG_BIG = -1000000000.0


def _select_kernel(x_ref, w_impT_ref, b_imp_ref, wk_ref, bk_ref, wv_ref,
                   bv_ref, idx_ref, kproj_ref, vproj_ref, sel_ref, *, k):
    S, D = x_ref.shape
    xb = x_ref[...]
    scores = jax.lax.dot_general(
        w_impT_ref[...], xb,
        dimension_numbers=(((1,), (1,)), ((), ())),
        preferred_element_type=jnp.float32) + b_imp_ref[...]
    lane_iota = jax.lax.broadcasted_iota(jnp.int32, (1, S), 1)
    k_iota = jax.lax.broadcasted_iota(jnp.int32, (1, k), 1)

    def body(i, carry):
        sc, acc = carry
        m = jnp.max(sc)
        cand = jnp.where(sc == m, lane_iota, S)
        idx = jnp.min(cand).astype(jnp.int32)
        acc = jnp.where(k_iota == i, idx, acc)
        sel_ref[pl.ds(i, 1), :] = x_ref[pl.ds(idx, 1), :]
        sc = jnp.where(lane_iota == idx, -jnp.inf, sc)
        return sc, acc

    acc0 = jnp.zeros((1, k), dtype=jnp.int32)
    _, acc = jax.lax.fori_loop(0, k, body, (scores, acc0))
    idx_ref[...] = acc
    sel = sel_ref[...].astype(jnp.bfloat16)
    kproj_ref[...] = (jnp.dot(sel, wk_ref[...],
                              preferred_element_type=jnp.float32)
                      + bk_ref[...]).astype(jnp.bfloat16)
    vproj_ref[...] = (jnp.dot(sel, wv_ref[...],
                              preferred_element_type=jnp.float32)
                      + bv_ref[...]).astype(jnp.bfloat16)


def _attn_kernel(x_ref, wq_ref, bq_ref, kproj_ref, vproj_ref, idx_ref,
                 wo_ref, bo_ref, out_ref, *, heads, dh, tile, k):
    t = pl.program_id(1)
    xt = x_ref[...].astype(jnp.bfloat16)
    q = jnp.dot(xt, wq_ref[...],
                preferred_element_type=jnp.float32) + bq_ref[...]
    q = q.astype(jnp.bfloat16)
    kp = kproj_ref[...]
    vp = vproj_ref[...]
    kpos = idx_ref[...]
    qpos = t * tile + jax.lax.broadcasted_iota(jnp.int32, (tile, 1), 0)
    mask = qpos >= kpos
    scale = 1.0 / jnp.sqrt(jnp.asarray(dh, dtype=jnp.float32))
    ctx = []
    for h in range(heads):
        sl = slice(h * dh, (h + 1) * dh)
        qh = q[:, sl]
        sh = jax.lax.dot_general(
            qh, kp[:, sl],
            dimension_numbers=(((1,), (1,)), ((), ())),
            preferred_element_type=jnp.float32) * scale
        sh = jnp.where(mask, sh, NEG_BIG)
        m = jnp.max(sh, axis=1, keepdims=True)
        e = jnp.exp(sh - m)
        attn = (e / jnp.sum(e, axis=1, keepdims=True)).astype(jnp.bfloat16)
        ctx.append(jnp.dot(attn, vp[:, sl],
                           preferred_element_type=jnp.float32))
    ctx = jnp.concatenate(ctx, axis=1).astype(jnp.bfloat16)
    out_ref[...] = jnp.dot(ctx, wo_ref[...],
                           preferred_element_type=jnp.float32) + bo_ref[...]


def kernel(x, W_imp, b_imp, Wq, bq, Wk, bk, Wv, bv, Wo, bo):
    B, S, D = x.shape
    HD = Wq.shape[1]
    heads, dh = 16, HD // 16
    k = min(TOP_K, S)
    tile = 512
    nt = S // tile

    w_impT = W_imp.T
    b_imp2 = b_imp.reshape(1, 1)
    bq2, bk2, bv2 = bq.reshape(1, HD), bk.reshape(1, HD), bv.reshape(1, HD)
    bo2 = bo.reshape(1, D)
    wq_b = Wq.astype(jnp.bfloat16)
    wk_b = Wk.astype(jnp.bfloat16)
    wv_b = Wv.astype(jnp.bfloat16)
    wo_b = Wo.astype(jnp.bfloat16)

    idx3, kproj, vproj = pl.pallas_call(
        functools.partial(_select_kernel, k=k),
        grid=(B,),
        in_specs=[
            pl.BlockSpec((None, S, D), lambda b: (b, 0, 0)),
            pl.BlockSpec((1, D), lambda b: (0, 0)),
            pl.BlockSpec((1, 1), lambda b: (0, 0)),
            pl.BlockSpec((D, HD), lambda b: (0, 0)),
            pl.BlockSpec((1, HD), lambda b: (0, 0)),
            pl.BlockSpec((D, HD), lambda b: (0, 0)),
            pl.BlockSpec((1, HD), lambda b: (0, 0)),
        ],
        out_specs=[
            pl.BlockSpec((None, 1, k), lambda b: (b, 0, 0)),
            pl.BlockSpec((None, k, HD), lambda b: (b, 0, 0)),
            pl.BlockSpec((None, k, HD), lambda b: (b, 0, 0)),
        ],
        out_shape=[
            jax.ShapeDtypeStruct((B, 1, k), jnp.int32),
            jax.ShapeDtypeStruct((B, k, HD), jnp.bfloat16),
            jax.ShapeDtypeStruct((B, k, HD), jnp.bfloat16),
        ],
        scratch_shapes=[pltpu.VMEM((k, D), jnp.float32)],
    )(x, w_impT, b_imp2, wk_b, bk2, wv_b, bv2)

    out = pl.pallas_call(
        functools.partial(_attn_kernel, heads=heads, dh=dh, tile=tile, k=k),
        grid=(B, nt),
        in_specs=[
            pl.BlockSpec((None, tile, D), lambda b, t: (b, t, 0)),
            pl.BlockSpec((D, HD), lambda b, t: (0, 0)),
            pl.BlockSpec((1, HD), lambda b, t: (0, 0)),
            pl.BlockSpec((None, k, HD), lambda b, t: (b, 0, 0)),
            pl.BlockSpec((None, k, HD), lambda b, t: (b, 0, 0)),
            pl.BlockSpec((None, 1, k), lambda b, t: (b, 0, 0)),
            pl.BlockSpec((HD, D), lambda b, t: (0, 0)),
            pl.BlockSpec((1, D), lambda b, t: (0, 0)),
        ],
        out_specs=pl.BlockSpec((None, tile, D), lambda b, t: (b, t, 0)),
        out_shape=jax.ShapeDtypeStruct((B, S, D), jnp.float32),
    )(x, wq_b, bq2, kproj, vproj, idx3, wo_b, bo2)

    return out, idx3.reshape(B, k)


# speedup vs baseline: 1.1506x; 1.1123x over previous
import functools

import jax
import jax.numpy as jnp
from jax.experimental import pallas as pl
from jax.experimental.pallas import tpu as pltpu

TOP_K = 64
NEG_BIG = -1000000000.0
PACK = 4


def _scores_kernel(x_ref, w_impT_ref, b_imp_ref, s_ref):
    s_ref[...] = jax.lax.dot_general(
        w_impT_ref[...], x_ref[...],
        dimension_numbers=(((1,), (1,)), ((), ())),
        preferred_element_type=jnp.float32) + b_imp_ref[...]


def _topk_kernel(s_ref, idx_ref, *, k):
    S = s_ref.shape[1]
    scores = s_ref[...]
    lane_iota = jax.lax.broadcasted_iota(jnp.int32, (1, S), 1)
    k_iota = jax.lax.broadcasted_iota(jnp.int32, (1, k), 1)

    def body(i, carry):
        sc, acc = carry
        m = jnp.max(sc)
        cand = jnp.where(sc == m, lane_iota, S)
        idx = jnp.min(cand).astype(jnp.int32)
        acc = jnp.where(k_iota == i, idx, acc)
        sc = jnp.where(lane_iota == idx, -jnp.inf, sc)
        return sc, acc

    acc0 = jnp.zeros((1, k), dtype=jnp.int32)
    _, acc = jax.lax.fori_loop(0, k, body, (scores, acc0))
    idx_ref[...] = acc


def _gather_kv_kernel(idx_sref, x_ref, wk_ref, bk_ref, wv_ref, bv_ref,
                      kblk_ref, vblk_ref, sel_ref, sem,
                      *, k, heads, dh):
    b = pl.program_id(0)
    copies = []
    for i in range(k):
        row = idx_sref[b * k + i]
        c = pltpu.make_async_copy(
            x_ref.at[b, pl.ds(row, 1), :], sel_ref.at[pl.ds(i, 1), :], sem)
        c.start()
        copies.append(c)
    for c in copies:
        c.wait()
    sel = sel_ref[...].astype(jnp.bfloat16)
    kp = (jnp.dot(sel, wk_ref[...], preferred_element_type=jnp.float32)
          + bk_ref[...]).astype(jnp.bfloat16)
    vp = (jnp.dot(sel, wv_ref[...], preferred_element_type=jnp.float32)
          + bv_ref[...]).astype(jnp.bfloat16)
    zero = jnp.zeros((k, dh), dtype=jnp.bfloat16)
    for g in range(heads // PACK):
        krows, vrows = [], []
        for hh in range(PACK):
            h = g * PACK + hh
            kb = [zero] * PACK
            vb = [zero] * PACK
            kb[hh] = kp[:, h * dh:(h + 1) * dh]
            vb[hh] = vp[:, h * dh:(h + 1) * dh]
            krows.append(jnp.concatenate(kb, axis=1))
            vrows.append(jnp.concatenate(vb, axis=1))
        kblk_ref[g] = jnp.concatenate(krows, axis=0)
        vblk_ref[g] = jnp.concatenate(vrows, axis=0)


def _attn_kernel(x_ref, wq_ref, bq_ref, kblk_ref, vblk_ref, idx_ref,
                 segdown_ref, segup_ref, wo_ref, bo_ref, out_ref,
                 *, heads, dh, tile, k):
    t = pl.program_id(1)
    xt = x_ref[...].astype(jnp.bfloat16)
    q = jnp.dot(xt, wq_ref[...],
                preferred_element_type=jnp.float32) + bq_ref[...]
    q = q.astype(jnp.bfloat16)
    grp = PACK * dh
    ngrp = heads // PACK
    scale = 1.0 / jnp.sqrt(jnp.asarray(dh, dtype=jnp.float32))
    sg = []
    for g in range(ngrp):
        sg.append(jax.lax.dot_general(
            q[:, g * grp:(g + 1) * grp], kblk_ref[g],
            dimension_numbers=(((1,), (1,)), ((), ())),
            preferred_element_type=jnp.float32))
    scores = jnp.concatenate(sg, axis=1) * scale
    kpos = idx_ref[...]
    kpos_rep = jnp.concatenate([kpos] * heads, axis=1)
    qpos = t * tile + jax.lax.broadcasted_iota(jnp.int32, (tile, 1), 0)
    scores = jnp.where(qpos >= kpos_rep, scores, NEG_BIG)
    m = jnp.max(scores, axis=1, keepdims=True)
    e = jnp.exp(scores - m)
    eb = e.astype(jnp.bfloat16)
    denom = jnp.dot(eb, segdown_ref[...],
                    preferred_element_type=jnp.float32)
    recip = (1.0 / denom).astype(jnp.bfloat16)
    rexp = jnp.dot(recip, segup_ref[...],
                   preferred_element_type=jnp.float32)
    attn = (e * rexp).astype(jnp.bfloat16)
    ctx = []
    for g in range(ngrp):
        ctx.append(jnp.dot(attn[:, g * grp:(g + 1) * grp], vblk_ref[g],
                           preferred_element_type=jnp.float32))
    ctxb = jnp.concatenate(ctx, axis=1).astype(jnp.bfloat16)
    out_ref[...] = jnp.dot(ctxb, wo_ref[...],
                           preferred_element_type=jnp.float32) + bo_ref[...]


def kernel(x, W_imp, b_imp, Wq, bq, Wk, bk, Wv, bv, Wo, bo):
    B, S, D = x.shape
    HD = Wq.shape[1]
    heads = 16
    dh = HD // heads
    k = min(TOP_K, S)
    tile = 512
    nt = S // tile
    stile = 512
    nst = S // stile
    ngrp = heads // PACK
    grp = PACK * k

    w_impT = W_imp.T
    b_imp2 = b_imp.reshape(1, 1)
    bq2, bk2, bv2 = bq.reshape(1, HD), bk.reshape(1, HD), bv.reshape(1, HD)
    bo2 = bo.reshape(1, D)
    wq_b = Wq.astype(jnp.bfloat16)
    wk_b = Wk.astype(jnp.bfloat16)
    wv_b = Wv.astype(jnp.bfloat16)
    wo_b = Wo.astype(jnp.bfloat16)
    head_of_lane = jnp.arange(heads * k) // k
    segdown = (head_of_lane[:, None] ==
               jnp.arange(heads)[None, :]).astype(jnp.bfloat16)
    segup = (jnp.arange(heads)[:, None] ==
             head_of_lane[None, :]).astype(jnp.bfloat16)

    scores = pl.pallas_call(
        _scores_kernel,
        grid=(B, nst),
        in_specs=[
            pl.BlockSpec((None, stile, D), lambda b, t: (b, t, 0)),
            pl.BlockSpec((1, D), lambda b, t: (0, 0)),
            pl.BlockSpec((1, 1), lambda b, t: (0, 0)),
        ],
        out_specs=pl.BlockSpec((None, 1, stile), lambda b, t: (b, 0, t)),
        out_shape=jax.ShapeDtypeStruct((B, 1, S), jnp.float32),
    )(x, w_impT, b_imp2)

    idx3 = pl.pallas_call(
        functools.partial(_topk_kernel, k=k),
        grid=(B,),
        in_specs=[pl.BlockSpec((None, 1, S), lambda b: (b, 0, 0))],
        out_specs=pl.BlockSpec((None, 1, k), lambda b: (b, 0, 0)),
        out_shape=jax.ShapeDtypeStruct((B, 1, k), jnp.int32),
    )(scores)

    kblk, vblk = pl.pallas_call(
        functools.partial(_gather_kv_kernel, k=k, heads=heads, dh=dh),
        grid_spec=pltpu.PrefetchScalarGridSpec(
            num_scalar_prefetch=1,
            grid=(B,),
            in_specs=[
                pl.BlockSpec(memory_space=pl.ANY),
                pl.BlockSpec((D, HD), lambda b, idx: (0, 0)),
                pl.BlockSpec((1, HD), lambda b, idx: (0, 0)),
                pl.BlockSpec((D, HD), lambda b, idx: (0, 0)),
                pl.BlockSpec((1, HD), lambda b, idx: (0, 0)),
            ],
            out_specs=[
                pl.BlockSpec((None, ngrp, grp, grp), lambda b, idx: (b, 0, 0, 0)),
                pl.BlockSpec((None, ngrp, grp, grp), lambda b, idx: (b, 0, 0, 0)),
            ],
            scratch_shapes=[
                pltpu.VMEM((k, D), jnp.float32),
                pltpu.SemaphoreType.DMA,
            ],
        ),
        out_shape=[
            jax.ShapeDtypeStruct((B, ngrp, grp, grp), jnp.bfloat16),
            jax.ShapeDtypeStruct((B, ngrp, grp, grp), jnp.bfloat16),
        ],
    )(idx3.reshape(B * k), x, wk_b, bk2, wv_b, bv2)

    out = pl.pallas_call(
        functools.partial(_attn_kernel, heads=heads, dh=dh, tile=tile, k=k),
        grid=(B, nt),
        in_specs=[
            pl.BlockSpec((None, tile, D), lambda b, t: (b, t, 0)),
            pl.BlockSpec((D, HD), lambda b, t: (0, 0)),
            pl.BlockSpec((1, HD), lambda b, t: (0, 0)),
            pl.BlockSpec((None, ngrp, grp, grp), lambda b, t: (b, 0, 0, 0)),
            pl.BlockSpec((None, ngrp, grp, grp), lambda b, t: (b, 0, 0, 0)),
            pl.BlockSpec((None, 1, k), lambda b, t: (b, 0, 0)),
            pl.BlockSpec((heads * k, heads), lambda b, t: (0, 0)),
            pl.BlockSpec((heads, heads * k), lambda b, t: (0, 0)),
            pl.BlockSpec((HD, D), lambda b, t: (0, 0)),
            pl.BlockSpec((1, D), lambda b, t: (0, 0)),
        ],
        out_specs=pl.BlockSpec((None, tile, D), lambda b, t: (b, t, 0)),
        out_shape=jax.ShapeDtypeStruct((B, S, D), jnp.float32),
    )(x, wq_b, bq2, kblk, vblk, idx3, segdown, segup, wo_b, bo2)

    return out, idx3.reshape(B, k)


# speedup vs baseline: 1.3043x; 1.1336x over previous
import functools

import jax
import jax.numpy as jnp
from jax.experimental import pallas as pl

TOP_K = 64
NEG_BIG = -1000000000.0
PACK = 4
SROW = 8


def _select_kernel(x_ref, w_impT_ref, b_imp_ref, wk_ref, bk_ref, wv_ref,
                   bv_ref, idx_ref, kblk_ref, vblk_ref, *, k, heads, dh):
    S, D = x_ref.shape
    scol = S // SROW
    w = w_impT_ref[...]
    chunks = []
    for r in range(SROW):
        chunks.append(jax.lax.dot_general(
            w, x_ref[r * scol:(r + 1) * scol, :],
            dimension_numbers=(((1,), (1,)), ((), ())),
            preferred_element_type=jnp.float32))
    scores = jnp.concatenate(chunks, axis=0) + b_imp_ref[...]
    lin = (jax.lax.broadcasted_iota(jnp.int32, (SROW, scol), 0) * scol +
           jax.lax.broadcasted_iota(jnp.int32, (SROW, scol), 1))
    k_iota = jax.lax.broadcasted_iota(jnp.int32, (1, k), 1)
    kcol_iota = jax.lax.broadcasted_iota(jnp.int32, (k, 1), 0)

    def body(i, carry):
        sc, acc, acc_col = carry
        m = jnp.max(sc)
        cand = jnp.where(sc == m, lin, S)
        idx = jnp.min(cand).astype(jnp.int32)
        acc = jnp.where(k_iota == i, idx, acc)
        acc_col = jnp.where(kcol_iota == i, idx, acc_col)
        sc = jnp.where(lin == idx, -jnp.inf, sc)
        return sc, acc, acc_col

    acc0 = jnp.zeros((1, k), dtype=jnp.int32)
    acc_col0 = jnp.zeros((k, 1), dtype=jnp.int32)
    _, acc, acc_col = jax.lax.fori_loop(0, k, body, (scores, acc0, acc_col0))
    idx_ref[...] = acc
    sel_lane = jax.lax.broadcasted_iota(jnp.int32, (k, S), 1)
    onehot = (sel_lane == acc_col).astype(jnp.bfloat16)
    xb = x_ref[...].astype(jnp.bfloat16)
    sel = jnp.dot(onehot, xb, preferred_element_type=jnp.float32)
    sel = sel.astype(jnp.bfloat16)
    kp = (jnp.dot(sel, wk_ref[...], preferred_element_type=jnp.float32)
          + bk_ref[...]).astype(jnp.bfloat16)
    vp = (jnp.dot(sel, wv_ref[...], preferred_element_type=jnp.float32)
          + bv_ref[...]).astype(jnp.bfloat16)
    zero = jnp.zeros((k, dh), dtype=jnp.bfloat16)
    for g in range(heads // PACK):
        krows, vrows = [], []
        for hh in range(PACK):
            h = g * PACK + hh
            kb = [zero] * PACK
            vb = [zero] * PACK
            kb[hh] = kp[:, h * dh:(h + 1) * dh]
            vb[hh] = vp[:, h * dh:(h + 1) * dh]
            krows.append(jnp.concatenate(kb, axis=1))
            vrows.append(jnp.concatenate(vb, axis=1))
        kblk_ref[g] = jnp.concatenate(krows, axis=0)
        vblk_ref[g] = jnp.concatenate(vrows, axis=0)


def _attn_kernel(x_ref, wq_ref, bq_ref, kblk_ref, vblk_ref, idx_ref,
                 segdown_ref, segup_ref, wo_ref, bo_ref, out_ref,
                 *, heads, dh, tile, k):
    t = pl.program_id(1)
    xt = x_ref[...].astype(jnp.bfloat16)
    q = jnp.dot(xt, wq_ref[...],
                preferred_element_type=jnp.float32) + bq_ref[...]
    q = q.astype(jnp.bfloat16)
    grp = PACK * dh
    ngrp = heads // PACK
    scale = 1.0 / jnp.sqrt(jnp.asarray(dh, dtype=jnp.float32))
    kpos = idx_ref[...]
    kpos_g = jnp.concatenate([kpos] * PACK, axis=1)
    qpos = t * tile + jax.lax.broadcasted_iota(jnp.int32, (tile, 1), 0)
    mask_g = qpos >= kpos_g
    sg = []
    for g in range(ngrp):
        s = jax.lax.dot_general(
            q[:, g * grp:(g + 1) * grp], kblk_ref[g],
            dimension_numbers=(((1,), (1,)), ((), ())),
            preferred_element_type=jnp.float32) * scale
        sg.append(jnp.where(mask_g, s, NEG_BIG))
    m = jnp.maximum(
        jnp.maximum(jnp.max(sg[0], axis=1, keepdims=True),
                    jnp.max(sg[1], axis=1, keepdims=True)),
        jnp.maximum(jnp.max(sg[2], axis=1, keepdims=True),
                    jnp.max(sg[3], axis=1, keepdims=True)))
    e = [jnp.exp(s - m) for s in sg]
    eb = [v.astype(jnp.bfloat16) for v in e]
    denom = jnp.dot(eb[0], segdown_ref[0], preferred_element_type=jnp.float32)
    for g in range(1, ngrp):
        denom += jnp.dot(eb[g], segdown_ref[g],
                         preferred_element_type=jnp.float32)
    recip = (1.0 / denom).astype(jnp.bfloat16)
    acc = None
    for g in range(ngrp):
        rexp = jnp.dot(recip, segup_ref[g],
                       preferred_element_type=jnp.float32)
        attn = (e[g] * rexp).astype(jnp.bfloat16)
        ctx = jnp.dot(attn, vblk_ref[g],
                      preferred_element_type=jnp.float32).astype(jnp.bfloat16)
        part = jnp.dot(ctx, wo_ref[g], preferred_element_type=jnp.float32)
        acc = part if acc is None else acc + part
    out_ref[...] = acc + bo_ref[...]


def kernel(x, W_imp, b_imp, Wq, bq, Wk, bk, Wv, bv, Wo, bo):
    B, S, D = x.shape
    HD = Wq.shape[1]
    heads = 16
    dh = HD // heads
    k = min(TOP_K, S)
    tile = 1024
    nt = S // tile
    ngrp = heads // PACK
    grp = PACK * k

    w_impT = W_imp.T
    b_imp2 = b_imp.reshape(1, 1)
    bq2, bk2, bv2 = bq.reshape(1, HD), bk.reshape(1, HD), bv.reshape(1, HD)
    bo2 = bo.reshape(1, D)
    wq_b = Wq.astype(jnp.bfloat16)
    wk_b = Wk.astype(jnp.bfloat16)
    wv_b = Wv.astype(jnp.bfloat16)
    wo_b = Wo.astype(jnp.bfloat16).reshape(ngrp, grp, D)
    lane_head = jnp.arange(grp) // k
    head_ids = jnp.arange(heads)
    segdown = jnp.stack([
        ((g * PACK + lane_head)[:, None] == head_ids[None, :])
        .astype(jnp.bfloat16) for g in range(ngrp)])
    segup = jnp.stack([
        (head_ids[:, None] == (g * PACK + lane_head)[None, :])
        .astype(jnp.bfloat16) for g in range(ngrp)])

    idx3, kblk, vblk = pl.pallas_call(
        functools.partial(_select_kernel, k=k, heads=heads, dh=dh),
        grid=(B,),
        in_specs=[
            pl.BlockSpec((None, S, D), lambda b: (b, 0, 0)),
            pl.BlockSpec((1, D), lambda b: (0, 0)),
            pl.BlockSpec((1, 1), lambda b: (0, 0)),
            pl.BlockSpec((D, HD), lambda b: (0, 0)),
            pl.BlockSpec((1, HD), lambda b: (0, 0)),
            pl.BlockSpec((D, HD), lambda b: (0, 0)),
            pl.BlockSpec((1, HD), lambda b: (0, 0)),
        ],
        out_specs=[
            pl.BlockSpec((None, 1, k), lambda b: (b, 0, 0)),
            pl.BlockSpec((None, ngrp, grp, grp), lambda b: (b, 0, 0, 0)),
            pl.BlockSpec((None, ngrp, grp, grp), lambda b: (b, 0, 0, 0)),
        ],
        out_shape=[
            jax.ShapeDtypeStruct((B, 1, k), jnp.int32),
            jax.ShapeDtypeStruct((B, ngrp, grp, grp), jnp.bfloat16),
            jax.ShapeDtypeStruct((B, ngrp, grp, grp), jnp.bfloat16),
        ],
    )(x, w_impT, b_imp2, wk_b, bk2, wv_b, bv2)

    out = pl.pallas_call(
        functools.partial(_attn_kernel, heads=heads, dh=dh, tile=tile, k=k),
        grid=(B, nt),
        in_specs=[
            pl.BlockSpec((None, tile, D), lambda b, t: (b, t, 0)),
            pl.BlockSpec((D, HD), lambda b, t: (0, 0)),
            pl.BlockSpec((1, HD), lambda b, t: (0, 0)),
            pl.BlockSpec((None, ngrp, grp, grp), lambda b, t: (b, 0, 0, 0)),
            pl.BlockSpec((None, ngrp, grp, grp), lambda b, t: (b, 0, 0, 0)),
            pl.BlockSpec((None, 1, k), lambda b, t: (b, 0, 0)),
            pl.BlockSpec((ngrp, grp, heads), lambda b, t: (0, 0, 0)),
            pl.BlockSpec((ngrp, heads, grp), lambda b, t: (0, 0, 0)),
            pl.BlockSpec((ngrp, grp, D), lambda b, t: (0, 0, 0)),
            pl.BlockSpec((1, D), lambda b, t: (0, 0)),
        ],
        out_specs=pl.BlockSpec((None, tile, D), lambda b, t: (b, t, 0)),
        out_shape=jax.ShapeDtypeStruct((B, S, D), jnp.float32),
    )(x, wq_b, bq2, kblk, vblk, idx3, segdown, segup, wo_b, bo2)

    return out, idx3.reshape(B, k)
